# baseline (speedup 1.0000x reference)
import jax
import jax.numpy as jnp
from jax.experimental import pallas as pl
from jax.experimental.pallas import tpu as pltpu

_BN = 1024
_K = 32
_NLAD = 15
_LADSH = 26


def _topk_mask(bits_ref, cnt_ref, samples_ref):
    b, n = bits_ref.shape

    cnts = n + cnt_ref[:, :_NLAD]
    lo0 = jnp.zeros((b, 1), jnp.int32)
    hi0 = jnp.full((b, 1), 0x3F800001, jnp.int32)
    ch0 = jnp.zeros((b, 1), jnp.int32)
    for j in range(_NLAD):
        cj = cnts[:, j:j + 1]
        lo0 = jnp.where(cj >= _K, (j + 1) << _LADSH, lo0)
    for j in reversed(range(_NLAD)):
        cj = cnts[:, j:j + 1]
        lt = cj < _K
        hi0 = jnp.where(lt, (j + 1) << _LADSH, hi0)
        ch0 = jnp.where(lt, cj, ch0)

    def q2(_, st):
        lo, hi, ch = st
        mid = lo + (hi - lo) // 2
        c = n + jnp.sum((bits_ref[...] - mid) >> 31, axis=1, keepdims=True)
        pred = c >= _K
        return (jnp.where(pred, mid, lo), jnp.where(pred, hi, mid),
                jnp.where(pred, ch, c))

    lo, _, ch = jax.lax.fori_loop(0, _LADSH, q2, (lo0, hi0, ch0))

    bits = bits_ref[...]
    gt = bits > lo
    m = _K - ch

    idx = jax.lax.broadcasted_iota(jnp.int32, (b, n), 1)

    def wcond(st):
        cnt, _ = st
        return jnp.any(cnt < m)

    def wbody(st):
        cnt, j = st
        active = cnt < m
        eq = bits_ref[...] == lo
        cand = jnp.where(eq & (idx > j), idx, n)
        jmin = jnp.min(cand, axis=1, keepdims=True)
        j = jnp.where(active, jmin, j)
        return cnt + active.astype(jnp.int32), j

    _, jf = jax.lax.while_loop(
        wcond, wbody,
        (jnp.zeros((b, 1), jnp.int32), jnp.full((b, 1), -1, jnp.int32)))

    eq = bits == lo
    samples_ref[...] = (gt | (eq & (idx <= jf))).astype(jnp.float32)


def _fused_body(gate_ref, x_ref, w1_ref, b1_ref, w2_ref, b2_ref, prior_ref,
                logits_ref, probs_ref, samples_ref, h_ref, bits_ref, cnt_ref):
    i = pl.program_id(0)

    @pl.when(i == 0)
    def _():
        h_ref[...] = jax.nn.relu(
            jnp.dot(x_ref[...], w1_ref[...],
                    preferred_element_type=jnp.float32) + b1_ref[...])

    base = jnp.dot(h_ref[...], w2_ref[...],
                   preferred_element_type=jnp.float32) + b2_ref[...]
    pc = jnp.clip(prior_ref[...], 1e-06, 1.0 - 1e-06)
    prior_logit = jnp.log(pc / (1.0 - pc))
    logits = base + gate_ref[0] * prior_logit
    probs = jax.nn.sigmoid(logits)
    logits_ref[...] = logits
    probs_ref[...] = probs
    blk_bits = jax.lax.bitcast_convert_type(probs, jnp.int32)
    bits_ref[:, pl.ds(i * _BN, _BN)] = blk_bits

    parts = [jnp.sum((blk_bits - ((j + 1) << _LADSH)) >> 31,
                     axis=1, keepdims=True) for j in range(_NLAD)]
    pc = jnp.concatenate(parts, axis=1)

    @pl.when(i == 0)
    def _():
        cnt_ref[:, :_NLAD] = pc

    @pl.when(i > 0)
    def _():
        cnt_ref[:, :_NLAD] = cnt_ref[:, :_NLAD] + pc

    @pl.when(i == pl.num_programs(0) - 1)
    def _():
        _topk_mask(bits_ref, cnt_ref, samples_ref)


def kernel(emb, emb_mask, attn_prior, W1, b1, W2, b2, prior_gate):
    del emb_mask
    B, _, D = emb.shape
    H = W1.shape[1]
    N = W2.shape[1]

    x = emb[:, 0, :]
    gate = jax.nn.sigmoid(prior_gate).reshape(1)
    b1_2d = b1.reshape(1, H)
    b2_2d = b2.reshape(1, N)

    grid = N // _BN
    logits, probs, samples = pl.pallas_call(
        _fused_body,
        grid=(grid,),
        in_specs=[
            pl.BlockSpec(memory_space=pltpu.SMEM),
            pl.BlockSpec((B, D), lambda i: (0, 0)),
            pl.BlockSpec((D, H), lambda i: (0, 0)),
            pl.BlockSpec((1, H), lambda i: (0, 0)),
            pl.BlockSpec((H, _BN), lambda i: (0, i)),
            pl.BlockSpec((1, _BN), lambda i: (0, i)),
            pl.BlockSpec((B, _BN), lambda i: (0, i)),
        ],
        out_specs=[
            pl.BlockSpec((B, _BN), lambda i: (0, i)),
            pl.BlockSpec((B, _BN), lambda i: (0, i)),
            pl.BlockSpec((B, N), lambda i: (0, 0)),
        ],
        out_shape=[
            jax.ShapeDtypeStruct((B, N), jnp.float32),
            jax.ShapeDtypeStruct((B, N), jnp.float32),
            jax.ShapeDtypeStruct((B, N), jnp.float32),
        ],
        scratch_shapes=[
            pltpu.VMEM((B, H), jnp.float32),
            pltpu.VMEM((B, N), jnp.int32),
            pltpu.VMEM((B, _NLAD + 1), jnp.int32),
        ],
        compiler_params=pltpu.CompilerParams(
            dimension_semantics=("arbitrary",)),
    )(gate, x, W1, b1_2d, W2, b2_2d, attn_prior)

    gate_detached = jax.nn.sigmoid(jax.lax.stop_gradient(prior_gate))
    return (samples, probs, logits, probs, gate_detached)


# speedup vs baseline: 1.0992x; 1.0992x over previous
import jax
import jax.numpy as jnp
from jax.experimental import pallas as pl
from jax.experimental.pallas import tpu as pltpu

_BN = 1024
_K = 32


def _topk_mask(bits_ref, samples_ref):
    b, n = bits_ref.shape

    def q2(_, st):
        lo, hi, ch = st
        mid = lo + (hi - lo) // 2
        c = n + jnp.sum((bits_ref[...] - mid) >> 31, axis=1, keepdims=True)
        pred = c >= _K
        return (jnp.where(pred, mid, lo), jnp.where(pred, hi, mid),
                jnp.where(pred, ch, c))

    st0 = (jnp.zeros((b, 1), jnp.int32),
           jnp.full((b, 1), 0x3F800001, jnp.int32),
           jnp.zeros((b, 1), jnp.int32))
    lo, _, ch = jax.lax.fori_loop(0, 30, q2, st0)

    bits = bits_ref[...]
    gt = bits > lo
    m = _K - ch

    idx = jax.lax.broadcasted_iota(jnp.int32, (b, n), 1)

    def wcond(st):
        cnt, _ = st
        return jnp.any(cnt < m)

    def wbody(st):
        cnt, j = st
        active = cnt < m
        eq = bits_ref[...] == lo
        cand = jnp.where(eq & (idx > j), idx, n)
        jmin = jnp.min(cand, axis=1, keepdims=True)
        j = jnp.where(active, jmin, j)
        return cnt + active.astype(jnp.int32), j

    _, jf = jax.lax.while_loop(
        wcond, wbody,
        (jnp.zeros((b, 1), jnp.int32), jnp.full((b, 1), -1, jnp.int32)))

    eq = bits == lo
    samples_ref[...] = (gt | (eq & (idx <= jf))).astype(jnp.float32)


def _fused_body(gate_ref, x_ref, w1_ref, b1_ref, w2_ref, b2_ref, prior_ref,
                logits_ref, probs_ref, samples_ref, h_ref, bits_ref):
    i = pl.program_id(0)

    @pl.when(i == 0)
    def _():
        h_ref[...] = jax.nn.relu(
            jnp.dot(x_ref[...], w1_ref[...],
                    preferred_element_type=jnp.float32) + b1_ref[...])

    base = jnp.dot(h_ref[...], w2_ref[...],
                   preferred_element_type=jnp.float32) + b2_ref[...]
    pc = jnp.clip(prior_ref[...], 1e-06, 1.0 - 1e-06)
    prior_logit = jnp.log(pc / (1.0 - pc))
    logits = base + gate_ref[0] * prior_logit
    probs = jax.nn.sigmoid(logits)
    logits_ref[...] = logits
    probs_ref[...] = probs
    bits_ref[:, pl.ds(i * _BN, _BN)] = jax.lax.bitcast_convert_type(
        probs, jnp.int32)

    @pl.when(i == pl.num_programs(0) - 1)
    def _():
        _topk_mask(bits_ref, samples_ref)


def kernel(emb, emb_mask, attn_prior, W1, b1, W2, b2, prior_gate):
    del emb_mask
    B, _, D = emb.shape
    H = W1.shape[1]
    N = W2.shape[1]

    x = emb[:, 0, :]
    gate = jax.nn.sigmoid(prior_gate).reshape(1)
    b1_2d = b1.reshape(1, H)
    b2_2d = b2.reshape(1, N)

    grid = N // _BN
    logits, probs, samples = pl.pallas_call(
        _fused_body,
        grid=(grid,),
        in_specs=[
            pl.BlockSpec(memory_space=pltpu.SMEM),
            pl.BlockSpec((B, D), lambda i: (0, 0)),
            pl.BlockSpec((D, H), lambda i: (0, 0)),
            pl.BlockSpec((1, H), lambda i: (0, 0)),
            pl.BlockSpec((H, _BN), lambda i: (0, i)),
            pl.BlockSpec((1, _BN), lambda i: (0, i)),
            pl.BlockSpec((B, _BN), lambda i: (0, i)),
        ],
        out_specs=[
            pl.BlockSpec((B, _BN), lambda i: (0, i)),
            pl.BlockSpec((B, _BN), lambda i: (0, i)),
            pl.BlockSpec((B, N), lambda i: (0, 0)),
        ],
        out_shape=[
            jax.ShapeDtypeStruct((B, N), jnp.float32),
            jax.ShapeDtypeStruct((B, N), jnp.float32),
            jax.ShapeDtypeStruct((B, N), jnp.float32),
        ],
        scratch_shapes=[
            pltpu.VMEM((B, H), jnp.float32),
            pltpu.VMEM((B, N), jnp.int32),
        ],
        compiler_params=pltpu.CompilerParams(
            dimension_semantics=("arbitrary",)),
    )(gate, x, W1, b1_2d, W2, b2_2d, attn_prior)

    gate_detached = jax.nn.sigmoid(jax.lax.stop_gradient(prior_gate))
    return (samples, probs, logits, probs, gate_detached)
